# baseline (speedup 1.0000x reference)
import jax
import jax.numpy as jnp
from jax.experimental import pallas as pl
from jax.experimental.pallas import tpu as pltpu

_B, _E, _A = 128, 4, 100000
_R = 8


def _body(q_ref, lp_ref, idx_ref):
    blk = q_ref[...]
    q = jnp.minimum(
        jnp.minimum(blk[:, 0:_A], blk[:, _A:2 * _A]),
        jnp.minimum(blk[:, 2 * _A:3 * _A], blk[:, 3 * _A:4 * _A]))
    mx = jnp.max(q, axis=1, keepdims=True)
    ids = jax.lax.broadcasted_iota(jnp.int32, (_R, _A), 1)
    best = jnp.min(jnp.where(q == mx, ids, jnp.int32(2147483647)),
                   axis=1, keepdims=True)
    shifted = q - mx
    lse = jnp.log(jnp.sum(jnp.exp(shifted), axis=1, keepdims=True))
    lp_ref[...] = shifted - lse
    idx_ref[...] = best


def kernel(Qs):
    q2 = Qs.reshape(_B, _E * _A)
    lp, idx = pl.pallas_call(
        _body,
        grid=(_B // _R,),
        in_specs=[pl.BlockSpec((_R, _E * _A), lambda i: (i, 0))],
        out_specs=[
            pl.BlockSpec((_R, _A), lambda i: (i, 0)),
            pl.BlockSpec((_R, 1), lambda i: (i, 0)),
        ],
        out_shape=[
            jax.ShapeDtypeStruct((_B, _A), jnp.float32),
            jax.ShapeDtypeStruct((_B, 1), jnp.int32),
        ],
    )(q2)
    return lp, idx[:, 0]
